# baseline (speedup 1.0000x reference)
import functools

import jax
import jax.numpy as jnp
from jax import lax
from jax.experimental import pallas as pl
from jax.experimental.pallas import tpu as pltpu
from jax.experimental.pallas import tpu_sc as plsc


@functools.lru_cache(maxsize=None)
def _make_kernel(B, L, D, V):
    info = plsc.get_sparse_core_info()
    NC, NS = info.num_cores, info.num_subcores
    NW = NC * NS
    bags_per_w = B // NW
    C = 8
    NB = 2
    nchunks = bags_per_w // C
    IDX = C * L
    NG = IDX // 16
    ND = D // 16
    GPR = D * 4 // 64
    inv_l = 1.0 / L

    mesh = plsc.VectorSubcoreMesh(core_axis_name="c", subcore_axis_name="s")

    @functools.partial(
        pl.kernel,
        mesh=mesh,
        compiler_params=pltpu.CompilerParams(use_tc_tiling_on_sc=False),
        out_type=jax.ShapeDtypeStruct((B, D), jnp.float32),
        scratch_types=[
            pltpu.VMEM((bags_per_w * L,), jnp.int32),
            pltpu.VMEM((NB, IDX * GPR, 16), jnp.float32),
            pltpu.VMEM((bags_per_w, D), jnp.float32),
        ] + [pltpu.SemaphoreType.DMA] * 2,
    )
    def k(idx_hbm, table_hbm, out_hbm, idx_v, rows_v, out_v, *sems):
        wid = lax.axis_index("s") * NC + lax.axis_index("c")
        bag_base = wid * bags_per_w
        pltpu.sync_copy(idx_hbm.at[pl.ds(bag_base * L, bags_per_w * L)],
                        idx_v)

        iota = lax.iota(jnp.int32, 16)
        rep0 = lax.shift_right_logical(iota, 2)
        reps = [rep0 + 4 * s for s in range(4)]
        mod4 = lax.bitwise_and(iota, 3)

        def gather_start(g, b):
            def group(i, _):
                vec = idx_v[pl.ds(g * IDX + i * 16, 16)]
                for s in range(4):
                    rows4 = vec.at[reps[s]].get(mode="promise_in_bounds")
                    gvec = rows4 * GPR + mod4
                    pltpu.async_copy(
                        table_hbm.at[gvec],
                        rows_v.at[b, pl.ds((i * 4 + s) * 16, 16)], sems[b])
                return 0

            lax.fori_loop(0, NG, group, 0, unroll=5)

        def gather_wait(b):
            pltpu.make_async_copy(
                table_hbm.at[pl.ds(0, IDX * GPR)], rows_v.at[b],
                sems[b]).wait()

        def compute(g, b):
            def bag_body(c, _):
                base = c * L

                def lbody(l, accs):
                    r = (base + l) * GPR
                    return tuple(accs[d] + rows_v[b, r + d, :]
                                 for d in range(ND))

                accs = lax.fori_loop(
                    0, L, lbody,
                    tuple(jnp.zeros((16,), jnp.float32) for _ in range(ND)),
                    unroll=10)
                row = g * C + c
                for d in range(ND):
                    out_v[row, pl.ds(d * 16, 16)] = accs[d] * inv_l
                return 0

            lax.fori_loop(0, C, bag_body, 0)

        for b in range(NB):
            gather_start(b, b)

        nfull = (nchunks // NB) * NB

        @pl.loop(0, nfull, step=NB)
        def _(j):
            for b in range(NB):
                g = j + b
                gather_wait(b)
                compute(g, b)

                @pl.when(g + NB < nchunks)
                def _():
                    gather_start(g + NB, b)

        for g in range(nfull, nchunks):
            b = g % NB
            gather_wait(b)
            compute(g, b)

        pltpu.sync_copy(out_v, out_hbm.at[pl.ds(bag_base, bags_per_w)])

    return k


def kernel(input_, weight):
    B, L = input_.shape
    V, D = weight.shape
    k = _make_kernel(B, L, D, V)
    return k(input_.reshape(-1), weight.reshape(V * 4, D // 4))


# speedup vs baseline: 1.1004x; 1.1004x over previous
import functools

import jax
import jax.numpy as jnp
from jax import lax
from jax.experimental import pallas as pl
from jax.experimental.pallas import tpu as pltpu
from jax.experimental.pallas import tpu_sc as plsc


@functools.lru_cache(maxsize=None)
def _make_kernel(B, L, D, V):
    info = plsc.get_sparse_core_info()
    NC, NS = info.num_cores, info.num_subcores
    NW = NC * NS
    bags_per_w = B // NW
    C = 4
    NB = 5
    nchunks = bags_per_w // C
    IDX = C * L
    ND = D // 16
    inv_l = 1.0 / L

    mesh = plsc.VectorSubcoreMesh(core_axis_name="c", subcore_axis_name="s")

    @functools.partial(
        pl.kernel,
        mesh=mesh,
        compiler_params=pltpu.CompilerParams(use_tc_tiling_on_sc=False),
        out_type=jax.ShapeDtypeStruct((B, D), jnp.float32),
        scratch_types=[
            pltpu.VMEM((bags_per_w * L,), jnp.int32),
            pltpu.VMEM((NB, IDX, D), jnp.float32),
            pltpu.VMEM((bags_per_w, D), jnp.float32),
        ] + [pltpu.SemaphoreType.DMA] * 5,
    )
    def k(idx_hbm, table_hbm, out_hbm, idx_v, rows_v, out_v, *sems):
        wid = lax.axis_index("s") * NC + lax.axis_index("c")
        bag_base = wid * bags_per_w
        pltpu.sync_copy(idx_hbm.at[pl.ds(bag_base * L, bags_per_w * L)],
                        idx_v)

        def gather_start(g, b):
            pltpu.async_copy(table_hbm.at[idx_v.at[pl.ds(g * IDX, IDX)]],
                             rows_v.at[b], sems[b])

        def gather_wait(b):
            pltpu.make_async_copy(
                table_hbm.at[idx_v.at[pl.ds(0, IDX)]], rows_v.at[b],
                sems[b]).wait()

        def compute(g, b):
            def bag_body(c, _):
                base = c * L

                def lbody(l, accs):
                    r = base + l
                    return tuple(accs[d] + rows_v[b, r, pl.ds(d * 16, 16)]
                                 for d in range(ND))

                accs = lax.fori_loop(
                    0, L, lbody,
                    tuple(jnp.zeros((16,), jnp.float32) for _ in range(ND)),
                    unroll=10)
                row = g * C + c
                for d in range(ND):
                    out_v[row, pl.ds(d * 16, 16)] = accs[d] * inv_l
                return 0

            lax.fori_loop(0, C, bag_body, 0)

        for b in range(NB):
            gather_start(b, b)

        nfull = (nchunks // NB) * NB

        @pl.loop(0, nfull, step=NB)
        def _(j):
            for b in range(NB):
                g = j + b
                gather_wait(b)
                compute(g, b)

                @pl.when(g + NB < nchunks)
                def _():
                    gather_start(g + NB, b)

        for g in range(nfull, nchunks):
            b = g % NB
            gather_wait(b)
            compute(g, b)

        pltpu.sync_copy(out_v, out_hbm.at[pl.ds(bag_base, bags_per_w)])

    return k


def kernel(input_, weight):
    B, L = input_.shape
    V, D = weight.shape
    k = _make_kernel(B, L, D, V)
    return k(input_.reshape(-1), weight)
